# baseline (speedup 1.0000x reference)
import functools

import jax
import jax.numpy as jnp
import numpy as np
from jax import lax
from jax.experimental import pallas as pl
from jax.experimental.pallas import tpu as pltpu
from jax.experimental.pallas import tpu_sc as plsc

N = 10000
E = 320000
D = 128
F2 = 2 * D

NC = 2
NS = 16
NW = NC * NS
EPW = E // NW
B = 80
CH = EPW // B
NP = 10240
NPS = NP // NS

_mesh = plsc.VectorSubcoreMesh(core_axis_name="c", subcore_axis_name="s")

_PERM = np.concatenate(
    [np.arange(16) + 32 * m for m in range(F2 // 32)]
    + [np.arange(16) + 32 * m + 16 for m in range(F2 // 32)]).astype(np.int32)

_SIGMA = np.zeros(D, dtype=np.int32)
for _m in range(D // 32):
    for _i in range(16):
        _SIGMA[16 * _m + _i] = 32 * _m + _i
        _SIGMA[64 + 16 * _m + _i] = 32 * _m + 16 + _i
_INV_SIGMA = np.argsort(_SIGMA).astype(np.int32)


def _pack_words(x):
    hi = x[:, :F2 // 2].astype(jnp.bfloat16)
    lo = x[:, F2 // 2:].astype(jnp.bfloat16)
    hu = lax.convert_element_type(lax.bitcast_convert_type(hi, jnp.uint16),
                                  jnp.uint32)
    lu = lax.convert_element_type(lax.bitcast_convert_type(lo, jnp.uint16),
                                  jnp.uint32)
    return lax.bitcast_convert_type((hu << 16) | lu, jnp.float32)


def _pack_words_dir(x, up):
    uh = lax.bitcast_convert_type(x[:, :F2 // 2], jnp.uint32)
    ul = lax.bitcast_convert_type(x[:, F2 // 2:], jnp.uint32)
    if up:
        uh = uh + jnp.uint32(0xFFFF)
        ul = ul + jnp.uint32(0xFFFF)
    w = (uh & jnp.uint32(0xFFFF0000)) | (ul >> 16)
    return lax.bitcast_convert_type(w, jnp.float32)


def _proj_body(nf_ref, ws_ref, wd_ref, ps_ref, pd0_ref, pd1_ref):
    x = nf_ref[...]
    ps_ref[...] = _pack_words(
        jnp.dot(x, ws_ref[...], preferred_element_type=jnp.float32))
    yd = jnp.dot(x, wd_ref[...], preferred_element_type=jnp.float32)
    pd0_ref[...] = _pack_words_dir(yd, False)
    pd1_ref[...] = _pack_words_dir(yd, True)


def _fuse_body(ef_ref, we_ref, ab_ref, lin_ref, part_ref):
    cmat = jnp.dot(ef_ref[...], we_ref[...], preferred_element_type=jnp.float32)
    hmask = jnp.int32(-65536)
    rnd = jnp.int32(32768)
    u = lax.bitcast_convert_type(ab_ref[...], jnp.int32)
    g0 = lax.bitcast_convert_type(u & hmask, jnp.float32)
    g1 = lax.bitcast_convert_type(u << 16, jnp.float32)
    v0 = g0 + cmat[:, :F2 // 2]
    v1 = g1 + cmat[:, F2 // 2:]
    w0 = (lax.bitcast_convert_type(v0, jnp.int32) + rnd) & hmask
    w1 = lax.shift_right_logical(
        lax.bitcast_convert_type(v1, jnp.int32) + rnd, 16)
    lin_ref[...] = lax.bitcast_convert_type(w0 | w1, jnp.float32)
    part_ref[...] = jnp.concatenate(
        [jnp.sum(v0, axis=0), jnp.sum(v1, axis=0),
         jnp.sum(v0 * v0, axis=0), jnp.sum(v1 * v1, axis=0)]).reshape(
             1, 1, 2 * F2)


def _make_gather_body(epw, bsz, nch):
  def _sc_gather_body(ps_hbm, pd0_hbm, pd1_hbm, src_hbm, dst_hbm,
                    ab_hbm,
                    sia_v, dia_v, a_v, b_v, w_v,
                    gsem0, gsem1, wsem0, wsem1):
    EPW, B, CH = epw, bsz, nch
    cid = lax.axis_index("c")
    sid = lax.axis_index("s")
    wid = sid * NC + cid
    base = wid * EPW
    gsem = (gsem0, gsem1)
    wsem = (wsem0, wsem1)
    ptbl = (pd0_hbm, pd1_hbm)

    pltpu.sync_copy(src_hbm.at[pl.ds(base, EPW)], sia_v)
    pltpu.sync_copy(dst_hbm.at[pl.ds(base, EPW)], dia_v)

    def issue(t, k):
        loc = t * B
        pltpu.async_copy(ps_hbm.at[sia_v.at[pl.ds(loc, B)]], a_v.at[k],
                         gsem[k])
        pltpu.async_copy(ptbl[k].at[dia_v.at[pl.ds(loc, B)]], b_v.at[k],
                         gsem[k])

    def drain_gather(k):
        pltpu.make_async_copy(ps_hbm.at[pl.ds(0, B)], a_v.at[k],
                              gsem[k]).wait()
        pltpu.make_async_copy(pd0_hbm.at[pl.ds(0, B)], b_v.at[k],
                              gsem[k]).wait()

    def drain_write(k):
        pltpu.make_async_copy(w_v.at[k], ab_hbm.at[pl.ds(base, B)],
                              wsem[k]).wait()

    def compute_chunk(t, k):
        def row_body(r, _unused):
            hmask = jnp.int32(-65536)
            rnd = jnp.int32(32768)
            for j in range(F2 // 32):
                wsl = pl.ds(16 * j, 16)
                ua = plsc.bitcast(a_v[k, r, wsl], jnp.int32)
                a0 = plsc.bitcast(ua & hmask, jnp.float32)
                a1 = plsc.bitcast(ua << 16, jnp.float32)
                ub = plsc.bitcast(b_v[k, r, wsl], jnp.int32)
                b0 = plsc.bitcast(ub & hmask, jnp.float32)
                b1 = plsc.bitcast(ub << 16, jnp.float32)
                v0 = a0 + b0
                v1 = a1 + b1
                w0 = (plsc.bitcast(v0, jnp.int32) + rnd) & hmask
                w1 = lax.shift_right_logical(
                    plsc.bitcast(v1, jnp.int32) + rnd, 16)
                w_v[k, r, wsl] = plsc.bitcast(w0 | w1, jnp.float32)
            return 0

        lax.fori_loop(0, B, row_body, 0)
        pltpu.async_copy(w_v.at[k], ab_hbm.at[pl.ds(base + t * B, B)],
                         wsem[k])

    issue(0, 0)
    issue(1, 1)
    drain_gather(0)
    compute_chunk(0, 0)

    def outer(g, _):
        for kk in range(2):
            t = 2 * g + 1 + kk
            k = (1 + kk) % 2
            kn = 1 - k

            @pl.when(t + 1 < CH)
            def _():
                issue(t + 1, kn)

            drain_gather(k)

            @pl.when(t >= 2)
            def _():
                drain_write(k)

            compute_chunk(t, k)
        return 0

    lax.fori_loop(0, (CH - 1) // 2, outer, 0)
    drain_write(0)
    drain_write(1)

  return _sc_gather_body


def _make_gather(ne, bsz):
    epw = ne // NW
    return functools.partial(
        pl.kernel,
        out_type=jax.ShapeDtypeStruct((ne, F2 // 2), jnp.float32),
        mesh=_mesh,
        scratch_types=[
            pltpu.VMEM((epw,), jnp.int32),
            pltpu.VMEM((epw,), jnp.int32),
            pltpu.VMEM((2, bsz, F2 // 2), jnp.float32),
            pltpu.VMEM((2, bsz, F2 // 2), jnp.float32),
            pltpu.VMEM((2, bsz, F2 // 2), jnp.float32),
            pltpu.SemaphoreType.DMA,
            pltpu.SemaphoreType.DMA,
            pltpu.SemaphoreType.DMA,
            pltpu.SemaphoreType.DMA,
        ],
        compiler_params=pltpu.CompilerParams(needs_layout_passes=False),
    )(_make_gather_body(epw, bsz, epw // bsz))


E0 = E // 2
E1 = E - E0
_sc_gather_0 = _make_gather(E0, 40)
_sc_gather_1 = _make_gather(E1, 40)


def _act_body(lin_ref, part_ref, part2_ref, g_ref, bt_ref, out_ref):
    part = part_ref[...]
    part2 = part2_ref[...]
    s = jnp.sum(part[:, :F2], axis=0) + jnp.sum(part2[:, :F2], axis=0)
    q = jnp.sum(part[:, F2:], axis=0) + jnp.sum(part2[:, F2:], axis=0)
    mu = s * (1.0 / E)
    var = q * (1.0 / E) - mu * mu
    inv = lax.rsqrt(var + 1e-5)
    scale = inv * g_ref[0]
    shift = bt_ref[0] - mu * scale
    u = lax.bitcast_convert_type(lin_ref[...], jnp.uint32)
    y_hi = lax.bitcast_convert_type(u & jnp.uint32(0xFFFF0000), jnp.float32)
    y_lo = lax.bitcast_convert_type(u << 16, jnp.float32)
    y = jnp.concatenate([y_hi, y_lo], axis=1)
    y = y * scale[None, :] + shift[None, :]
    h = D // 2
    y1 = jnp.concatenate([y[:, :h], y[:, D:D + h]], axis=1)
    y2 = jnp.concatenate([y[:, h:D], y[:, D + h:]], axis=1)
    sig = jax.nn.sigmoid(y1)
    sp = jnp.maximum(y2, 0.0) + jnp.log1p(jnp.exp(-jnp.abs(y2)))
    out_ref[...] = sig * sp


def _make_scatter_body(epw, bsz, nch):
  def _sc_scatter_body(msg_hbm, dst_hbm, init_hbm, agg_hbm,
                     di0_v, di1_v, m_v, acc_sh, isem0, isem1, msem0, msem1):
    EPW, B, CH = epw, bsz, nch
    cid = lax.axis_index("c")
    sid = lax.axis_index("s")
    wid = sid * NC + cid
    base = wid * EPW
    rows = pl.ds(sid * NPS, NPS)
    di = (di0_v, di1_v)
    isem = (isem0, isem1)
    msem = (msem0, msem1)

    pltpu.sync_copy(init_hbm.at[cid, rows], acc_sh.at[rows])

    def issue(t, k):
        off = base + t * B
        pltpu.async_copy(dst_hbm.at[pl.ds(off, B)], di[k], isem[k])
        pltpu.async_copy(msg_hbm.at[pl.ds(off, B)], m_v.at[k], msem[k])

    def drain(k):
        pltpu.make_async_copy(dst_hbm.at[pl.ds(base, B)], di[k],
                              isem[k]).wait()
        pltpu.make_async_copy(msg_hbm.at[pl.ds(base, B)], m_v.at[k],
                              msem[k]).wait()

    plsc.subcore_barrier()
    issue(0, 0)
    issue(1, 1)
    drain(0)
    pltpu.sync_copy(m_v.at[0], acc_sh.at[di[0]], add=True)

    def outer(g, _):
        for kk in range(2):
            t = 2 * g + 1 + kk
            k = (1 + kk) % 2
            kn = 1 - k

            @pl.when(t + 1 < CH)
            def _():
                issue(t + 1, kn)

            drain(k)
            pltpu.sync_copy(m_v.at[k], acc_sh.at[di[k]], add=True)
        return 0

    lax.fori_loop(0, (CH - 1) // 2, outer, 0)
    plsc.subcore_barrier()
    pltpu.sync_copy(acc_sh.at[rows], agg_hbm.at[cid, rows])

  return _sc_scatter_body


def _make_scatter(ne, bsz):
    epw = ne // NW
    return functools.partial(
        pl.kernel,
        out_type=jax.ShapeDtypeStruct((NC, NP, D), jnp.float32),
        mesh=_mesh,
        scratch_types=[
            pltpu.VMEM((bsz,), jnp.int32),
            pltpu.VMEM((bsz,), jnp.int32),
            pltpu.VMEM((2, bsz, D), jnp.float32),
            pltpu.VMEM_SHARED((NP, D), jnp.float32),
            pltpu.SemaphoreType.DMA,
            pltpu.SemaphoreType.DMA,
            pltpu.SemaphoreType.DMA,
            pltpu.SemaphoreType.DMA,
        ],
    )(_make_scatter_body(epw, bsz, epw // bsz))


_sc_scatter_0 = _make_scatter(E0, 40)
_sc_scatter_1 = _make_scatter(E1, 40)


def _node_body(agg_ref, nf_ref, g_ref, bt_ref, out_ref):
    x = agg_ref[...]
    agg = x[0, :N] + x[1, :N]
    mu = jnp.mean(agg, axis=0)
    var = jnp.mean(agg * agg, axis=0) - mu * mu
    inv = lax.rsqrt(var + 1e-5)
    scale = inv * g_ref[0]
    shift = bt_ref[0] - mu * scale
    out_ref[...] = jax.nn.sigmoid(agg * scale[None, :] + shift[None, :]
                                  + nf_ref[...])


def kernel(node_feats, edge_feats, edge_index,
           mlpt_W, mlpt_b, mlpt_gamma, mlpt_beta,
           gate_W, gate_b, gate_gamma, gate_beta,
           node_gamma, node_beta):
    f32 = jnp.float32
    ws = jnp.concatenate([mlpt_W[:D], gate_W[:D]], axis=1)[:, _PERM]
    wd = jnp.concatenate([mlpt_W[D:2 * D], gate_W[D:2 * D]], axis=1)[:, _PERM]
    we = jnp.concatenate([mlpt_W[2 * D:], gate_W[2 * D:]], axis=1)[:, _PERM]
    g2 = jnp.concatenate([mlpt_gamma, gate_gamma])[_PERM].reshape(1, F2)
    bt2 = jnp.concatenate([mlpt_beta, gate_beta])[_PERM].reshape(1, F2)
    src = edge_index[0]
    dst = edge_index[1]

    nb = 2000
    ps, pd0, pd1 = pl.pallas_call(
        _proj_body,
        grid=(N // nb,),
        in_specs=[pl.BlockSpec((nb, D), lambda i: (i, 0)),
                  pl.BlockSpec((D, F2), lambda i: (0, 0)),
                  pl.BlockSpec((D, F2), lambda i: (0, 0))],
        out_specs=[pl.BlockSpec((nb, F2 // 2), lambda i: (i, 0)),
                   pl.BlockSpec((nb, F2 // 2), lambda i: (i, 0)),
                   pl.BlockSpec((nb, F2 // 2), lambda i: (i, 0))],
        out_shape=[jax.ShapeDtypeStruct((N, F2 // 2), f32),
                   jax.ShapeDtypeStruct((N, F2 // 2), f32),
                   jax.ShapeDtypeStruct((N, F2 // 2), f32)],
    )(node_feats, ws, wd)

    src0, src1 = src[:E0], src[E0:]
    dst0, dst1 = dst[:E0], dst[E0:]

    ab0 = _sc_gather_0(ps, pd0, pd1, src0, dst0)
    ab1 = _sc_gather_1(ps, pd0, pd1, src1, dst1)

    eb = 16000

    def fuse(ef_h, ab_h, ne):
        nblk = ne // eb
        lin, part = pl.pallas_call(
            _fuse_body,
            grid=(nblk,),
            in_specs=[pl.BlockSpec((eb, D), lambda i: (i, 0)),
                      pl.BlockSpec((D, F2), lambda i: (0, 0)),
                      pl.BlockSpec((eb, F2 // 2), lambda i: (i, 0))],
            out_specs=[pl.BlockSpec((eb, F2 // 2), lambda i: (i, 0)),
                       pl.BlockSpec((1, 1, 2 * F2), lambda i: (i, 0, 0))],
            out_shape=[jax.ShapeDtypeStruct((ne, F2 // 2), f32),
                       jax.ShapeDtypeStruct((nblk, 1, 2 * F2), f32)],
        )(ef_h, we, ab_h)
        return lin, part.reshape(nblk, 2 * F2)

    lin0, part0 = fuse(edge_feats[:E0], ab0, E0)
    lin1, part1 = fuse(edge_feats[E0:], ab1, E1)
    nb0 = E0 // eb
    nb1 = E1 // eb

    def act(lin_h, ne):
        nblk = ne // eb
        return pl.pallas_call(
            _act_body,
            grid=(nblk,),
            in_specs=[pl.BlockSpec((eb, F2 // 2), lambda i: (i, 0)),
                      pl.BlockSpec((nb0, 2 * F2), lambda i: (0, 0)),
                      pl.BlockSpec((nb1, 2 * F2), lambda i: (0, 0)),
                      pl.BlockSpec((1, F2), lambda i: (0, 0)),
                      pl.BlockSpec((1, F2), lambda i: (0, 0))],
            out_specs=pl.BlockSpec((eb, D), lambda i: (i, 0)),
            out_shape=jax.ShapeDtypeStruct((ne, D), f32),
        )(lin_h, part0, part1, g2, bt2)

    msg0 = act(lin0, E0)
    msg1 = act(lin1, E1)

    aggp0 = _sc_scatter_0(msg0, dst0, jnp.zeros((NC, NP, D), f32))
    aggp1 = _sc_scatter_1(msg1, dst1, aggp0)

    out_nodes = pl.pallas_call(
        _node_body,
        in_specs=[pl.BlockSpec((NC, NP, D), lambda: (0, 0, 0)),
                  pl.BlockSpec((N, D), lambda: (0, 0)),
                  pl.BlockSpec((1, D), lambda: (0, 0)),
                  pl.BlockSpec((1, D), lambda: (0, 0))],
        out_specs=pl.BlockSpec((N, D), lambda: (0, 0)),
        out_shape=jax.ShapeDtypeStruct((N, D), f32),
    )(aggp1, node_feats[:, _SIGMA], node_gamma[_SIGMA].reshape(1, D),
      node_beta[_SIGMA].reshape(1, D))

    return out_nodes[:, _INV_SIGMA], edge_feats


# speedup vs baseline: 1.0019x; 1.0019x over previous
import functools

import jax
import jax.numpy as jnp
import numpy as np
from jax import lax
from jax.experimental import pallas as pl
from jax.experimental.pallas import tpu as pltpu
from jax.experimental.pallas import tpu_sc as plsc

N = 10000
E = 320000
D = 128
F2 = 2 * D

NC = 2
NS = 16
NW = NC * NS
EPW = E // NW
B = 80
CH = EPW // B
NP = 10240
NPS = NP // NS

_mesh = plsc.VectorSubcoreMesh(core_axis_name="c", subcore_axis_name="s")

_PERM = np.concatenate(
    [np.arange(16) + 32 * m for m in range(F2 // 32)]
    + [np.arange(16) + 32 * m + 16 for m in range(F2 // 32)]).astype(np.int32)

_SIGMA = np.zeros(D, dtype=np.int32)
for _m in range(D // 32):
    for _i in range(16):
        _SIGMA[16 * _m + _i] = 32 * _m + _i
        _SIGMA[64 + 16 * _m + _i] = 32 * _m + 16 + _i
_INV_SIGMA = np.argsort(_SIGMA).astype(np.int32)


def _pack_words(x):
    hi = x[:, :F2 // 2].astype(jnp.bfloat16)
    lo = x[:, F2 // 2:].astype(jnp.bfloat16)
    hu = lax.convert_element_type(lax.bitcast_convert_type(hi, jnp.uint16),
                                  jnp.uint32)
    lu = lax.convert_element_type(lax.bitcast_convert_type(lo, jnp.uint16),
                                  jnp.uint32)
    return lax.bitcast_convert_type((hu << 16) | lu, jnp.float32)


def _pack_words_dir(x, up):
    uh = lax.bitcast_convert_type(x[:, :F2 // 2], jnp.uint32)
    ul = lax.bitcast_convert_type(x[:, F2 // 2:], jnp.uint32)
    if up:
        uh = uh + jnp.uint32(0xFFFF)
        ul = ul + jnp.uint32(0xFFFF)
    w = (uh & jnp.uint32(0xFFFF0000)) | (ul >> 16)
    return lax.bitcast_convert_type(w, jnp.float32)


def _proj_body(nf_ref, ws_ref, wd_ref, ps_ref, pd0_ref, pd1_ref):
    x = nf_ref[...]
    ps_ref[...] = _pack_words(
        jnp.dot(x, ws_ref[...], preferred_element_type=jnp.float32))
    yd = jnp.dot(x, wd_ref[...], preferred_element_type=jnp.float32)
    pd0_ref[...] = _pack_words_dir(yd, False)
    pd1_ref[...] = _pack_words_dir(yd, True)


def _fuse_body(ef_ref, we_ref, ab_ref, lin_ref, part_ref):
    cmat = jnp.dot(ef_ref[...], we_ref[...], preferred_element_type=jnp.float32)
    hmask = jnp.int32(-65536)
    rnd = jnp.int32(32768)
    u = lax.bitcast_convert_type(ab_ref[...], jnp.int32)
    g0 = lax.bitcast_convert_type(u & hmask, jnp.float32)
    g1 = lax.bitcast_convert_type(u << 16, jnp.float32)
    v0 = g0 + cmat[:, :F2 // 2]
    v1 = g1 + cmat[:, F2 // 2:]
    w0 = (lax.bitcast_convert_type(v0, jnp.int32) + rnd) & hmask
    w1 = lax.shift_right_logical(
        lax.bitcast_convert_type(v1, jnp.int32) + rnd, 16)
    lin_ref[...] = lax.bitcast_convert_type(w0 | w1, jnp.float32)
    part_ref[...] = jnp.concatenate(
        [jnp.sum(v0, axis=0), jnp.sum(v1, axis=0),
         jnp.sum(v0 * v0, axis=0), jnp.sum(v1 * v1, axis=0)]).reshape(
             1, 1, 2 * F2)


def _make_gather_body(epw, bsz, nch):
  def _sc_gather_body(ps_hbm, pd0_hbm, pd1_hbm, src_hbm, dst_hbm,
                    ab_hbm,
                    sia_v, dia_v, a_v, b_v, w_v,
                    gsem0, gsem1, wsem0, wsem1):
    EPW, B, CH = epw, bsz, nch
    cid = lax.axis_index("c")
    sid = lax.axis_index("s")
    wid = sid * NC + cid
    base = wid * EPW
    gsem = (gsem0, gsem1)
    wsem = (wsem0, wsem1)
    ptbl = (pd0_hbm, pd1_hbm)

    pltpu.sync_copy(src_hbm.at[pl.ds(base, EPW)], sia_v)
    pltpu.sync_copy(dst_hbm.at[pl.ds(base, EPW)], dia_v)

    def issue(t, k):
        loc = t * B
        pltpu.async_copy(ps_hbm.at[sia_v.at[pl.ds(loc, B)]], a_v.at[k],
                         gsem[k])
        pltpu.async_copy(ptbl[k].at[dia_v.at[pl.ds(loc, B)]], b_v.at[k],
                         gsem[k])

    def drain_gather(k):
        pltpu.make_async_copy(ps_hbm.at[pl.ds(0, B)], a_v.at[k],
                              gsem[k]).wait()
        pltpu.make_async_copy(pd0_hbm.at[pl.ds(0, B)], b_v.at[k],
                              gsem[k]).wait()

    def drain_write(k):
        pltpu.make_async_copy(w_v.at[k], ab_hbm.at[pl.ds(base, B)],
                              wsem[k]).wait()

    def compute_chunk(t, k):
        def row_body(r, _unused):
            hmask = jnp.int32(-65536)
            rnd = jnp.int32(32768)
            for j in range(F2 // 32):
                wsl = pl.ds(16 * j, 16)
                ua = plsc.bitcast(a_v[k, r, wsl], jnp.int32)
                a0 = plsc.bitcast(ua & hmask, jnp.float32)
                a1 = plsc.bitcast(ua << 16, jnp.float32)
                ub = plsc.bitcast(b_v[k, r, wsl], jnp.int32)
                b0 = plsc.bitcast(ub & hmask, jnp.float32)
                b1 = plsc.bitcast(ub << 16, jnp.float32)
                v0 = a0 + b0
                v1 = a1 + b1
                w0 = (plsc.bitcast(v0, jnp.int32) + rnd) & hmask
                w1 = lax.shift_right_logical(
                    plsc.bitcast(v1, jnp.int32) + rnd, 16)
                w_v[k, r, wsl] = plsc.bitcast(w0 | w1, jnp.float32)
            return 0

        lax.fori_loop(0, B, row_body, 0)
        pltpu.async_copy(w_v.at[k], ab_hbm.at[pl.ds(base + t * B, B)],
                         wsem[k])

    issue(0, 0)
    issue(1, 1)
    drain_gather(0)
    compute_chunk(0, 0)

    def outer(g, _):
        for kk in range(2):
            t = 2 * g + 1 + kk
            k = (1 + kk) % 2
            kn = 1 - k

            @pl.when(t + 1 < CH)
            def _():
                issue(t + 1, kn)

            drain_gather(k)

            @pl.when(t >= 2)
            def _():
                drain_write(k)

            compute_chunk(t, k)
        return 0

    lax.fori_loop(0, (CH - 1) // 2, outer, 0)
    drain_write(0)
    drain_write(1)

  return _sc_gather_body


def _make_gather(ne, bsz):
    epw = ne // NW
    return functools.partial(
        pl.kernel,
        out_type=jax.ShapeDtypeStruct((ne, F2 // 2), jnp.float32),
        mesh=_mesh,
        scratch_types=[
            pltpu.VMEM((epw,), jnp.int32),
            pltpu.VMEM((epw,), jnp.int32),
            pltpu.VMEM((2, bsz, F2 // 2), jnp.float32),
            pltpu.VMEM((2, bsz, F2 // 2), jnp.float32),
            pltpu.VMEM((2, bsz, F2 // 2), jnp.float32),
            pltpu.SemaphoreType.DMA,
            pltpu.SemaphoreType.DMA,
            pltpu.SemaphoreType.DMA,
            pltpu.SemaphoreType.DMA,
        ],
        compiler_params=pltpu.CompilerParams(needs_layout_passes=False),
    )(_make_gather_body(epw, bsz, epw // bsz))


E0 = E // 2
E1 = E - E0
_sc_gather_0 = _make_gather(E0, 40)
_sc_gather_1 = _make_gather(E1, 40)


def _act_body(lin_ref, part_ref, part2_ref, g_ref, bt_ref, out_ref):
    part = part_ref[...]
    part2 = part2_ref[...]
    s = jnp.sum(part[:, :F2], axis=0) + jnp.sum(part2[:, :F2], axis=0)
    q = jnp.sum(part[:, F2:], axis=0) + jnp.sum(part2[:, F2:], axis=0)
    mu = s * (1.0 / E)
    var = q * (1.0 / E) - mu * mu
    inv = lax.rsqrt(var + 1e-5)
    scale = inv * g_ref[0]
    shift = bt_ref[0] - mu * scale
    u = lax.bitcast_convert_type(lin_ref[...], jnp.uint32)
    y_hi = lax.bitcast_convert_type(u & jnp.uint32(0xFFFF0000), jnp.float32)
    y_lo = lax.bitcast_convert_type(u << 16, jnp.float32)
    y = jnp.concatenate([y_hi, y_lo], axis=1)
    y = y * scale[None, :] + shift[None, :]
    h = D // 2
    y1 = jnp.concatenate([y[:, :h], y[:, D:D + h]], axis=1)
    y2 = jnp.concatenate([y[:, h:D], y[:, D + h:]], axis=1)
    sig = jax.nn.sigmoid(y1)
    sp = jnp.maximum(y2, 0.0) + jnp.log1p(jnp.exp(-jnp.abs(y2)))
    out_ref[...] = sig * sp


def _make_scatter_body(epw, bsz, nch):
  def _sc_scatter_body(msg_hbm, dst_hbm, init_hbm, agg_hbm,
                     di0_v, di1_v, m_v, acc_sh, isem0, isem1, msem0, msem1):
    EPW, B, CH = epw, bsz, nch
    cid = lax.axis_index("c")
    sid = lax.axis_index("s")
    wid = sid * NC + cid
    base = wid * EPW
    rows = pl.ds(sid * NPS, NPS)
    di = (di0_v, di1_v)
    isem = (isem0, isem1)
    msem = (msem0, msem1)

    pltpu.sync_copy(init_hbm.at[cid, rows], acc_sh.at[rows])

    def issue(t, k):
        off = base + t * B
        pltpu.async_copy(dst_hbm.at[pl.ds(off, B)], di[k], isem[k])
        pltpu.async_copy(msg_hbm.at[pl.ds(off, B)], m_v.at[k], msem[k])

    def drain(k):
        pltpu.make_async_copy(dst_hbm.at[pl.ds(base, B)], di[k],
                              isem[k]).wait()
        pltpu.make_async_copy(msg_hbm.at[pl.ds(base, B)], m_v.at[k],
                              msem[k]).wait()

    plsc.subcore_barrier()
    issue(0, 0)
    issue(1, 1)
    drain(0)
    pltpu.sync_copy(m_v.at[0], acc_sh.at[di[0]], add=True)

    def outer(g, _):
        for kk in range(2):
            t = 2 * g + 1 + kk
            k = (1 + kk) % 2
            kn = 1 - k

            @pl.when(t + 1 < CH)
            def _():
                issue(t + 1, kn)

            drain(k)
            pltpu.sync_copy(m_v.at[k], acc_sh.at[di[k]], add=True)
        return 0

    lax.fori_loop(0, (CH - 1) // 2, outer, 0)
    plsc.subcore_barrier()
    pltpu.sync_copy(acc_sh.at[rows], agg_hbm.at[cid, rows])

  return _sc_scatter_body


def _make_scatter(ne, bsz):
    epw = ne // NW
    return functools.partial(
        pl.kernel,
        out_type=jax.ShapeDtypeStruct((NC, NP, D), jnp.float32),
        mesh=_mesh,
        scratch_types=[
            pltpu.VMEM((bsz,), jnp.int32),
            pltpu.VMEM((bsz,), jnp.int32),
            pltpu.VMEM((2, bsz, D), jnp.float32),
            pltpu.VMEM_SHARED((NP, D), jnp.float32),
            pltpu.SemaphoreType.DMA,
            pltpu.SemaphoreType.DMA,
            pltpu.SemaphoreType.DMA,
            pltpu.SemaphoreType.DMA,
        ],
    )(_make_scatter_body(epw, bsz, epw // bsz))


_sc_scatter_0 = _make_scatter(E0, 40)
_sc_scatter_1 = _make_scatter(E1, 40)


def _node_body(agg_ref, nf_ref, g_ref, bt_ref, out_ref):
    x = agg_ref[...]
    agg = x[0, :N] + x[1, :N]
    mu = jnp.mean(agg, axis=0)
    var = jnp.mean(agg * agg, axis=0) - mu * mu
    inv = lax.rsqrt(var + 1e-5)
    scale = inv * g_ref[0]
    shift = bt_ref[0] - mu * scale
    out_ref[...] = jax.nn.sigmoid(agg * scale[None, :] + shift[None, :]
                                  + nf_ref[...])


def kernel(node_feats, edge_feats, edge_index,
           mlpt_W, mlpt_b, mlpt_gamma, mlpt_beta,
           gate_W, gate_b, gate_gamma, gate_beta,
           node_gamma, node_beta):
    f32 = jnp.float32
    ws = jnp.concatenate([mlpt_W[:D], gate_W[:D]], axis=1)[:, _PERM]
    wd = jnp.concatenate([mlpt_W[D:2 * D], gate_W[D:2 * D]], axis=1)[:, _PERM]
    we = jnp.concatenate([mlpt_W[2 * D:], gate_W[2 * D:]], axis=1)[:, _PERM]
    g2 = jnp.concatenate([mlpt_gamma, gate_gamma])[_PERM].reshape(1, F2)
    bt2 = jnp.concatenate([mlpt_beta, gate_beta])[_PERM].reshape(1, F2)
    src = edge_index[0]
    dst = edge_index[1]

    nb = 2000
    ps, pd0, pd1 = pl.pallas_call(
        _proj_body,
        grid=(N // nb,),
        in_specs=[pl.BlockSpec((nb, D), lambda i: (i, 0)),
                  pl.BlockSpec((D, F2), lambda i: (0, 0)),
                  pl.BlockSpec((D, F2), lambda i: (0, 0))],
        out_specs=[pl.BlockSpec((nb, F2 // 2), lambda i: (i, 0)),
                   pl.BlockSpec((nb, F2 // 2), lambda i: (i, 0)),
                   pl.BlockSpec((nb, F2 // 2), lambda i: (i, 0))],
        out_shape=[jax.ShapeDtypeStruct((N, F2 // 2), f32),
                   jax.ShapeDtypeStruct((N, F2 // 2), f32),
                   jax.ShapeDtypeStruct((N, F2 // 2), f32)],
    )(node_feats, ws, wd)

    src0, src1 = src[:E0], src[E0:]
    dst0, dst1 = dst[:E0], dst[E0:]

    ab0 = _sc_gather_0(ps, pd0, pd1, src0, dst0)
    ab1 = _sc_gather_1(ps, pd0, pd1, src1, dst1)

    eb = 8000

    def fuse(ef_h, ab_h, ne):
        nblk = ne // eb
        lin, part = pl.pallas_call(
            _fuse_body,
            grid=(nblk,),
            in_specs=[pl.BlockSpec((eb, D), lambda i: (i, 0)),
                      pl.BlockSpec((D, F2), lambda i: (0, 0)),
                      pl.BlockSpec((eb, F2 // 2), lambda i: (i, 0))],
            out_specs=[pl.BlockSpec((eb, F2 // 2), lambda i: (i, 0)),
                       pl.BlockSpec((1, 1, 2 * F2), lambda i: (i, 0, 0))],
            out_shape=[jax.ShapeDtypeStruct((ne, F2 // 2), f32),
                       jax.ShapeDtypeStruct((nblk, 1, 2 * F2), f32)],
        )(ef_h, we, ab_h)
        return lin, part.reshape(nblk, 2 * F2)

    lin0, part0 = fuse(edge_feats[:E0], ab0, E0)
    lin1, part1 = fuse(edge_feats[E0:], ab1, E1)
    nb0 = E0 // eb
    nb1 = E1 // eb

    def act(lin_h, ne):
        nblk = ne // eb
        return pl.pallas_call(
            _act_body,
            grid=(nblk,),
            in_specs=[pl.BlockSpec((eb, F2 // 2), lambda i: (i, 0)),
                      pl.BlockSpec((nb0, 2 * F2), lambda i: (0, 0)),
                      pl.BlockSpec((nb1, 2 * F2), lambda i: (0, 0)),
                      pl.BlockSpec((1, F2), lambda i: (0, 0)),
                      pl.BlockSpec((1, F2), lambda i: (0, 0))],
            out_specs=pl.BlockSpec((eb, D), lambda i: (i, 0)),
            out_shape=jax.ShapeDtypeStruct((ne, D), f32),
        )(lin_h, part0, part1, g2, bt2)

    msg0 = act(lin0, E0)
    msg1 = act(lin1, E1)

    aggp0 = _sc_scatter_0(msg0, dst0, jnp.zeros((NC, NP, D), f32))
    aggp1 = _sc_scatter_1(msg1, dst1, aggp0)

    out_nodes = pl.pallas_call(
        _node_body,
        in_specs=[pl.BlockSpec((NC, NP, D), lambda: (0, 0, 0)),
                  pl.BlockSpec((N, D), lambda: (0, 0)),
                  pl.BlockSpec((1, D), lambda: (0, 0)),
                  pl.BlockSpec((1, D), lambda: (0, 0))],
        out_specs=pl.BlockSpec((N, D), lambda: (0, 0)),
        out_shape=jax.ShapeDtypeStruct((N, D), f32),
    )(aggp1, node_feats[:, _SIGMA], node_gamma[_SIGMA].reshape(1, D),
      node_beta[_SIGMA].reshape(1, D))

    return out_nodes[:, _INV_SIGMA], edge_feats
